# baseline (speedup 1.0000x reference)
import jax
import jax.numpy as jnp
from jax import lax
from jax.experimental import pallas as pl
from jax.experimental.pallas import tpu as pltpu
from jax.experimental.pallas import tpu_sc as plsc

B = 16384
L = 20
V = 1000000
D = 32

NC = 2
NS = 16
NW = NC * NS

COLS_FULL = (V // 128) * 128
TAIL = V - COLS_FULL
TCHUNK = 512
TOTAL_TCHUNKS = COLS_FULL // TCHUNK
TSTEPS = 63
TNBUF = 3
TWORDS = TCHUNK * D

WORKERS_PER_FEATURE = NW // 2
BAGS_PER_WORKER = B // WORKERS_PER_FEATURE
CHUNK = 64
CHUNK_IDS = CHUNK * L
IDS_PER_GATHER = 128
GATHERS = CHUNK_IDS // IDS_PER_GATHER
CHUNKS = BAGS_PER_WORKER // CHUNK
NBUF = 2


def _sc_relayout_kernel(tt_hbm, tail_hbm, out_hbm, cbuf, tbuf, vtail, gsem, osem):
    wid = lax.axis_index("s") * NC + lax.axis_index("c")

    def chunk_copy(k, slot):
        return pltpu.make_async_copy(
            tt_hbm.at[:, pl.ds(k * TCHUNK, TCHUNK)],
            cbuf.at[pl.ds(slot * D, D), :],
            gsem.at[slot],
        )

    def fire_in(t, slot):
        k = wid + 32 * t

        @pl.when(k < TOTAL_TCHUNKS)
        def _():
            chunk_copy(k, slot).start()

    def wait_in(t, slot):
        k = wid + 32 * t

        @pl.when(k < TOTAL_TCHUNKS)
        def _():
            chunk_copy(k, slot).wait()

    def wait_out(t, slot):
        k = wid + 32 * t

        @pl.when(jnp.logical_and(t >= 0, k < TOTAL_TCHUNKS))
        def _():
            pltpu.make_async_copy(
                tbuf.at[pl.ds(slot * TWORDS, TWORDS)],
                out_hbm.at[pl.ds(k * TWORDS, TWORDS)],
                osem.at[slot],
            ).wait()

    def transpose_fire_out(t, slot):
        k = wid + 32 * t

        @pl.when(k < TOTAL_TCHUNKS)
        def _():
            cb = cbuf.at[pl.ds(slot * D, D), :]
            tb = tbuf.at[pl.ds(slot * TWORDS, TWORDS)]
            iota = tuple(range(16))
            iota_v = jnp.arange(16, dtype=jnp.int32)

            rows = {}
            dsts = {}
            for d0 in (0, 16):
                for o in range(16):
                    rot = (iota_v + o) % 16
                    rows[o, d0] = rot + d0
                    dsts[o, d0] = iota_v * D + rot + d0

            @plsc.parallel_loop(0, TCHUNK // 16, unroll=2)
            def _blk(cb16):
                c0 = cb16 * 16
                colv = iota_v + c0
                dstb = jnp.full((16,), slot * TWORDS + c0 * D, dtype=jnp.int32)
                for d0 in (0, 16):
                    for o in range(16):
                        vals = plsc.load_gather(cb, [rows[o, d0], colv])
                        plsc.store_scatter(tbuf, [dsts[o, d0] + dstb], vals)

            pltpu.async_copy(
                tb, out_hbm.at[pl.ds(k * TWORDS, TWORDS)], osem.at[slot]
            )

    fire_in(0, 0)
    fire_in(1, 1)

    def ring_body(g, carry):
        for b2 in range(TNBUF):
            t = TNBUF * g + b2
            fire_in(t + 2, (b2 + 2) % TNBUF)
            wait_in(t, b2)
            wait_out(t - TNBUF, b2)
            transpose_fire_out(t, b2)
        return carry

    lax.fori_loop(0, TSTEPS // TNBUF, ring_body, 0)
    for b2 in range(TNBUF):
        wait_out(TSTEPS - TNBUF + b2, b2)

    @pl.when(wid == 0)
    def _():
        pltpu.sync_copy(tail_hbm, vtail)
        pltpu.sync_copy(vtail, out_hbm.at[pl.ds(COLS_FULL * D, TAIL * D)])


def _sc_pool_kernel(v1_hbm, v2_hbm, table_hbm, out_hbm, idx_v, rows_v, out_v, gsem):
    wid = lax.axis_index("s") * NC + lax.axis_index("c")
    fid = wid // WORKERS_PER_FEATURE
    bag0 = (wid % WORKERS_PER_FEATURE) * BAGS_PER_WORKER

    def run_feature(ids_hbm, col):
        def fire(i, slot):
            base = (bag0 + i * CHUNK) * L
            pltpu.sync_copy(ids_hbm.at[pl.ds(base, CHUNK_IDS)], idx_v.at[slot])
            for j in range(GATHERS):
                pltpu.async_copy(
                    table_hbm.at[idx_v.at[slot].at[pl.ds(j * IDS_PER_GATHER, IDS_PER_GATHER)]],
                    rows_v.at[slot].at[pl.ds(j * IDS_PER_GATHER, IDS_PER_GATHER)],
                    gsem.at[slot],
                )

        def drain(slot):
            for j in range(GATHERS):
                pltpu.make_async_copy(
                    table_hbm.at[idx_v.at[slot].at[pl.ds(j * IDS_PER_GATHER, IDS_PER_GATHER)]],
                    rows_v.at[slot].at[pl.ds(j * IDS_PER_GATHER, IDS_PER_GATHER)],
                    gsem.at[slot],
                ).wait()

        def reduce_and_write(i, slot):
            rv = rows_v.at[slot]

            @plsc.parallel_loop(0, CHUNK, unroll=2)
            def _bag(b):
                r0 = b * L
                lo = rv[r0, pl.ds(0, 16)]
                hi = rv[r0, pl.ds(16, 16)]
                for k in range(1, L):
                    lo = lo + rv[r0 + k, pl.ds(0, 16)]
                    hi = hi + rv[r0 + k, pl.ds(16, 16)]
                out_v[b, pl.ds(0, 16)] = lo
                out_v[b, pl.ds(16, 16)] = hi

            pltpu.sync_copy(
                out_v, out_hbm.at[pl.ds(bag0 + i * CHUNK, CHUNK), pl.ds(col, D)]
            )

        fire(0, 0)

        def pair_body(g, carry):
            for b in range(NBUF):
                i = 2 * g + b

                @pl.when(i + 1 < CHUNKS)
                def _():
                    fire(i + 1, 1 - b)

                drain(b)
                reduce_and_write(i, b)
            return carry

        lax.fori_loop(0, CHUNKS // NBUF, pair_body, 0)

    @pl.when(fid == 0)
    def _():
        run_feature(v1_hbm, 0)

    @pl.when(fid == 1)
    def _():
        run_feature(v2_hbm, D)


def kernel(values_f1, lengths_f1, values_f2, lengths_f2, table):
    del lengths_f1, lengths_f2
    mesh = plsc.VectorSubcoreMesh(
        core_axis_name="c", subcore_axis_name="s", num_cores=NC, num_subcores=NS
    )

    tt = table.T
    tail = table[COLS_FULL:, :].reshape(TAIL * D)
    relayout = pl.kernel(
        _sc_relayout_kernel,
        out_type=jax.ShapeDtypeStruct((V * D,), jnp.float32),
        mesh=mesh,
        scratch_types=[
            pltpu.VMEM((TNBUF * D, TCHUNK), jnp.float32),
            pltpu.VMEM((TNBUF * TWORDS,), jnp.float32),
            pltpu.VMEM((TAIL * D,), jnp.float32),
            pltpu.SemaphoreType.DMA((TNBUF,)),
            pltpu.SemaphoreType.DMA((TNBUF,)),
        ],
        compiler_params=pltpu.CompilerParams(
            use_tc_tiling_on_sc=True, needs_layout_passes=False
        ),
    )
    flat = relayout(tt, tail)
    table_rm = flat.reshape(V, D)

    pool = pl.kernel(
        _sc_pool_kernel,
        out_type=jax.ShapeDtypeStruct((B, 2 * D), jnp.float32),
        mesh=mesh,
        scratch_types=[
            pltpu.VMEM((NBUF, CHUNK_IDS), jnp.int32),
            pltpu.VMEM((NBUF, CHUNK_IDS, D), jnp.float32),
            pltpu.VMEM((CHUNK, D), jnp.float32),
            pltpu.SemaphoreType.DMA((NBUF,)),
        ],
        compiler_params=pltpu.CompilerParams(use_tc_tiling_on_sc=False),
    )
    return pool(values_f1, values_f2, table_rm)


# speedup vs baseline: 1.0656x; 1.0656x over previous
import jax
import jax.numpy as jnp
from jax import lax
from jax.experimental import pallas as pl
from jax.experimental.pallas import tpu as pltpu
from jax.experimental.pallas import tpu_sc as plsc

B = 16384
L = 20
V = 1000000
D = 32

NC = 2
NS = 16
NW = NC * NS

COLS_FULL = (V // 128) * 128
TAIL = V - COLS_FULL
TCHUNK = 896
TOTAL_TCHUNKS = COLS_FULL // TCHUNK
TSTEPS = 36
TWORDS = TCHUNK * D

WORKERS_PER_FEATURE = NW // 2
BAGS_PER_WORKER = B // WORKERS_PER_FEATURE
CHUNK = 64
CHUNK_IDS = CHUNK * L
IDS_PER_GATHER = 128
GATHERS = CHUNK_IDS // IDS_PER_GATHER
CHUNKS = BAGS_PER_WORKER // CHUNK
NBUF = 2


def _sc_relayout_kernel(tt_hbm, tail_hbm, out_hbm, cbuf, tbuf, vtail, gsem, osem):
    wid = lax.axis_index("s") * NC + lax.axis_index("c")

    def chunk_copy(k, slot):
        return pltpu.make_async_copy(
            tt_hbm.at[:, pl.ds(k * TCHUNK, TCHUNK)],
            cbuf.at[pl.ds(slot * D, D), :],
            gsem.at[slot],
        )

    def fire_in(t, slot):
        k = wid + 32 * t

        @pl.when(k < TOTAL_TCHUNKS)
        def _():
            chunk_copy(k, slot).start()

    def wait_in(t, slot):
        k = wid + 32 * t

        @pl.when(k < TOTAL_TCHUNKS)
        def _():
            chunk_copy(k, slot).wait()

    def wait_out(t, slot):
        k = wid + 32 * t

        @pl.when(jnp.logical_and(t >= 0, k < TOTAL_TCHUNKS))
        def _():
            pltpu.make_async_copy(
                tbuf.at[pl.ds(slot * TWORDS, TWORDS)],
                out_hbm.at[pl.ds(k * TWORDS, TWORDS)],
                osem.at[slot],
            ).wait()

    def transpose_fire_out(t, slot):
        k = wid + 32 * t

        @pl.when(k < TOTAL_TCHUNKS)
        def _():
            cb = cbuf.at[pl.ds(slot * D, D), :]
            tb = tbuf.at[pl.ds(slot * TWORDS, TWORDS)]
            iota = tuple(range(16))
            iota_v = jnp.arange(16, dtype=jnp.int32)

            rows = {}
            dsts = {}
            for d0 in (0, 16):
                for o in range(16):
                    rot = (iota_v + o) % 16
                    rows[o, d0] = rot + d0
                    dsts[o, d0] = iota_v * D + rot + d0

            @plsc.parallel_loop(0, TCHUNK // 16, unroll=2)
            def _blk(cb16):
                c0 = cb16 * 16
                colv = iota_v + c0
                dstb = jnp.full((16,), slot * TWORDS + c0 * D, dtype=jnp.int32)
                for d0 in (0, 16):
                    for o in range(16):
                        vals = plsc.load_gather(cb, [rows[o, d0], colv])
                        plsc.store_scatter(tbuf, [dsts[o, d0] + dstb], vals)

            pltpu.async_copy(
                tb, out_hbm.at[pl.ds(k * TWORDS, TWORDS)], osem.at[slot]
            )

    fire_in(0, 0)

    def pair_body(g, carry):
        for b2 in range(2):
            t = 2 * g + b2
            fire_in(t + 1, 1 - b2)
            wait_in(t, b2)
            wait_out(t - 2, b2)
            transpose_fire_out(t, b2)
        return carry

    lax.fori_loop(0, TSTEPS // 2, pair_body, 0)
    wait_out(TSTEPS - 2, 0)
    wait_out(TSTEPS - 1, 1)

    @pl.when(wid == 0)
    def _():
        pltpu.sync_copy(tail_hbm, vtail)
        pltpu.sync_copy(vtail, out_hbm.at[pl.ds(COLS_FULL * D, TAIL * D)])


def _sc_pool_kernel(v1_hbm, v2_hbm, table_hbm, out_hbm, idx_v, rows_v, out_v, gsem):
    wid = lax.axis_index("s") * NC + lax.axis_index("c")
    fid = wid // WORKERS_PER_FEATURE
    bag0 = (wid % WORKERS_PER_FEATURE) * BAGS_PER_WORKER

    def run_feature(ids_hbm, col):
        def fire(i, slot):
            base = (bag0 + i * CHUNK) * L
            pltpu.sync_copy(ids_hbm.at[pl.ds(base, CHUNK_IDS)], idx_v.at[slot])
            for j in range(GATHERS):
                pltpu.async_copy(
                    table_hbm.at[idx_v.at[slot].at[pl.ds(j * IDS_PER_GATHER, IDS_PER_GATHER)]],
                    rows_v.at[slot].at[pl.ds(j * IDS_PER_GATHER, IDS_PER_GATHER)],
                    gsem.at[slot],
                )

        def drain(slot):
            for j in range(GATHERS):
                pltpu.make_async_copy(
                    table_hbm.at[idx_v.at[slot].at[pl.ds(j * IDS_PER_GATHER, IDS_PER_GATHER)]],
                    rows_v.at[slot].at[pl.ds(j * IDS_PER_GATHER, IDS_PER_GATHER)],
                    gsem.at[slot],
                ).wait()

        def reduce_and_write(i, slot):
            rv = rows_v.at[slot]

            @plsc.parallel_loop(0, CHUNK, unroll=2)
            def _bag(b):
                r0 = b * L
                lo = rv[r0, pl.ds(0, 16)]
                hi = rv[r0, pl.ds(16, 16)]
                for k in range(1, L):
                    lo = lo + rv[r0 + k, pl.ds(0, 16)]
                    hi = hi + rv[r0 + k, pl.ds(16, 16)]
                out_v[b, pl.ds(0, 16)] = lo
                out_v[b, pl.ds(16, 16)] = hi

            pltpu.sync_copy(
                out_v, out_hbm.at[pl.ds(bag0 + i * CHUNK, CHUNK), pl.ds(col, D)]
            )

        fire(0, 0)

        def pair_body(g, carry):
            for b in range(NBUF):
                i = 2 * g + b

                @pl.when(i + 1 < CHUNKS)
                def _():
                    fire(i + 1, 1 - b)

                drain(b)
                reduce_and_write(i, b)
            return carry

        lax.fori_loop(0, CHUNKS // NBUF, pair_body, 0)

    @pl.when(fid == 0)
    def _():
        run_feature(v1_hbm, 0)

    @pl.when(fid == 1)
    def _():
        run_feature(v2_hbm, D)


def kernel(values_f1, lengths_f1, values_f2, lengths_f2, table):
    del lengths_f1, lengths_f2
    mesh = plsc.VectorSubcoreMesh(
        core_axis_name="c", subcore_axis_name="s", num_cores=NC, num_subcores=NS
    )

    tt = table.T
    tail = table[COLS_FULL:, :].reshape(TAIL * D)
    relayout = pl.kernel(
        _sc_relayout_kernel,
        out_type=jax.ShapeDtypeStruct((V * D,), jnp.float32),
        mesh=mesh,
        scratch_types=[
            pltpu.VMEM((NBUF * D, TCHUNK), jnp.float32),
            pltpu.VMEM((NBUF * TWORDS,), jnp.float32),
            pltpu.VMEM((TAIL * D,), jnp.float32),
            pltpu.SemaphoreType.DMA((NBUF,)),
            pltpu.SemaphoreType.DMA((NBUF,)),
        ],
        compiler_params=pltpu.CompilerParams(
            use_tc_tiling_on_sc=True, needs_layout_passes=False
        ),
    )
    flat = relayout(tt, tail)
    table_rm = flat.reshape(V, D)

    pool = pl.kernel(
        _sc_pool_kernel,
        out_type=jax.ShapeDtypeStruct((B, 2 * D), jnp.float32),
        mesh=mesh,
        scratch_types=[
            pltpu.VMEM((NBUF, CHUNK_IDS), jnp.int32),
            pltpu.VMEM((NBUF, CHUNK_IDS, D), jnp.float32),
            pltpu.VMEM((CHUNK, D), jnp.float32),
            pltpu.SemaphoreType.DMA((NBUF,)),
        ],
        compiler_params=pltpu.CompilerParams(use_tc_tiling_on_sc=False),
    )
    return pool(values_f1, values_f2, table_rm)
